# baseline (speedup 1.0000x reference)
import functools

import jax
import jax.numpy as jnp
from jax import lax
from jax.experimental import pallas as pl
from jax.experimental.pallas import tpu as pltpu
from jax.experimental.pallas import tpu_sc as plsc
import numpy as np

_N = 10000
_D = 256
_HALF = 128
_E = 160000
_L = 8
_ALPHA = 0.1
_THETA = 0.5

_NT = 16
_EB = 128
_B = 79
_EPAD = _NT * _B * _EB

_PR = 5000
_TROWS = 5120
_ZRT = _TROWS // _NT
_DUMMY = 5056
_COPY = 312

_R = 1000

_MESH = plsc.VectorSubcoreMesh(core_axis_name="c", subcore_axis_name="s")


def _zero_table(table, zbuf, s):
    zb = s * _ZRT
    for k in range(_ZRT // _EB):
        pltpu.sync_copy(zbuf, table.at[pl.ds(zb + k * _EB, _EB)])
    zrem = _ZRT % _EB
    if zrem:
        pltpu.sync_copy(zbuf.at[pl.ds(0, zrem)],
                        table.at[pl.ds(zb + (_ZRT // _EB) * _EB, zrem)])


def _copy_out_pass(table, buf, out_hbm, c, s, p):
    def copy_out(lr0, nr):
        pltpu.sync_copy(table.at[pl.ds(lr0, nr)], buf.at[pl.ds(0, nr)])
        pltpu.sync_copy(
            buf.at[pl.ds(0, nr)],
            out_hbm.at[pl.ds(p * _PR + lr0, nr), pl.ds(c * _HALF, _HALF)])

    ob = s * _COPY
    for k in range(_COPY // _EB):
        copy_out(ob + k * _EB, _EB)
    crem = _COPY % _EB
    if crem:
        copy_out(ob + (_COPY // _EB) * _EB, crem)

    @pl.when(s == _NT - 1)
    def _():
        copy_out(_NT * _COPY, _PR - _NT * _COPY)


@functools.partial(
    pl.kernel,
    out_type=jax.ShapeDtypeStruct((_N, _D), jnp.float32),
    mesh=_MESH,
    scratch_types=[
        pltpu.VMEM((_B, _EB), jnp.int32),
        pltpu.VMEM((_B, _EB), jnp.int32),
        pltpu.VMEM((1, _EB), jnp.int32),
        pltpu.VMEM((_EB, _HALF), jnp.float32),
        pltpu.VMEM((_EB, _HALF), jnp.float32),
        pltpu.VMEM((_EB, _HALF), jnp.float32),
        pltpu.VMEM((_EB, _HALF), jnp.float32),
        pltpu.SemaphoreType.DMA,
        pltpu.SemaphoreType.DMA,
        pltpu.SemaphoreType.DMA,
        pltpu.VMEM_SHARED((_TROWS, _HALF), jnp.float32),
    ],
)
def _sc_scatter(ht0, ht1, rowi, coli, zeros_hbm, out_hbm,
                row_v, col_v, loc_v, buf0, buf1, buf2, zbuf, g0, g1, g2, table):
    c = lax.axis_index("c")
    s = lax.axis_index("s")
    bufs = (buf0, buf1, buf2)
    gsems = (g0, g1, g2)

    pltpu.sync_copy(rowi.at[s], row_v)
    pltpu.sync_copy(coli.at[s], col_v)
    pltpu.sync_copy(zeros_hbm, zbuf)

    for p in range(2):
        _zero_table(table, zbuf, s)
        plsc.subcore_barrier()

        def run(h_hbm, p=p):
            def gather_start(k, t):
                pltpu.async_copy(h_hbm.at[row_v.at[t]], bufs[k], gsems[k])

            def compute_loc(t):
                for g in range(_EB // 16):
                    cv = col_v[t, pl.ds(g * 16, 16)]
                    loc = cv - (p * _PR)
                    ok = (loc >= 0) & (loc < _PR)
                    loc_v[0, pl.ds(g * 16, 16)] = jnp.where(ok, loc, _DUMMY)

            def finish(k, t):
                compute_loc(t)
                pltpu.make_async_copy(zeros_hbm, bufs[k], gsems[k]).wait()
                pltpu.sync_copy(bufs[k], table.at[loc_v.at[0]], add=True)

            gather_start(0, 0)
            gather_start(1, 1)

            def steady(o, carry):
                for b in range(3):
                    t = 3 * o + b
                    gather_start((b + 2) % 3, t + 2)
                    finish(b, t)
                return carry

            lax.fori_loop(0, (_B - 2) // 3, steady, 0)
            for t in range(_B - ((_B - 2) % 3) - 2, _B):
                if t + 2 < _B:
                    gather_start((t + 2) % 3, t + 2)
                finish(t % 3, t)

        @pl.when(c == 0)
        def _(run=run):
            run(ht0)

        @pl.when(c == 1)
        def _(run=run):
            run(ht1)

        plsc.subcore_barrier()
        _copy_out_pass(table, buf0, out_hbm, c, s, p)
        plsc.subcore_barrier()


@functools.partial(
    pl.kernel,
    out_type=jax.ShapeDtypeStruct((_N, _D), jnp.float32),
    mesh=_MESH,
    scratch_types=[
        pltpu.VMEM((_B, _EB), jnp.int32),
        pltpu.VMEM((1, _EB), jnp.int32),
        pltpu.VMEM((_EB, _HALF), jnp.float32),
        pltpu.VMEM((_EB, _HALF), jnp.float32),
        pltpu.VMEM_SHARED((_TROWS, _HALF), jnp.float32),
    ],
)
def _sc_count(coli, ones_hbm, zeros_hbm, out_hbm,
              col_v, loc_v, obuf, zbuf, table):
    c = lax.axis_index("c")
    s = lax.axis_index("s")

    pltpu.sync_copy(coli.at[s], col_v)
    pltpu.sync_copy(zeros_hbm, zbuf)
    pltpu.sync_copy(ones_hbm, obuf)

    for p in range(2):
        _zero_table(table, zbuf, s)
        plsc.subcore_barrier()

        def step(j, carry, p=p):
            for g in range(_EB // 16):
                cv = col_v[j, pl.ds(g * 16, 16)]
                loc = cv - (p * _PR)
                ok = (loc >= 0) & (loc < _PR)
                loc_v[0, pl.ds(g * 16, 16)] = jnp.where(ok, loc, _DUMMY)
            pltpu.sync_copy(obuf, table.at[loc_v.at[0]], add=True)
            return carry
        lax.fori_loop(0, _B, step, 0)

        plsc.subcore_barrier()
        _copy_out_pass(table, obuf, out_hbm, c, s, p)
        pltpu.sync_copy(ones_hbm, obuf)
        plsc.subcore_barrier()
        plsc.subcore_barrier()


def _in_body(x_ref, w_ref, b_ref, o_ref):
    o_ref[...] = jnp.maximum(
        jnp.dot(x_ref[...], w_ref[...], preferred_element_type=jnp.float32)
        + b_ref[...], 0.0)


_tc_in = pl.pallas_call(
    _in_body,
    grid=(_N // _R,),
    in_specs=[
        pl.BlockSpec((_R, _D), lambda i: (i, 0)),
        pl.BlockSpec((_D, _D), lambda i: (0, 0)),
        pl.BlockSpec((1, _D), lambda i: (0, 0)),
    ],
    out_specs=pl.BlockSpec((_R, _D), lambda i: (i, 0)),
    out_shape=jax.ShapeDtypeStruct((_N, _D), jnp.float32),
)

_HALF_SPEC = pl.BlockSpec((_R, _HALF), lambda i: (i, 0))
_HALF_SHAPE = jax.ShapeDtypeStruct((_N, _HALF), jnp.float32)


def _prep_body(cnt_ref, h0_ref, dis_ref, o0_ref, o1_ref):
    dis = lax.rsqrt(1.0 + cnt_ref[:, :1])
    dis_ref[...] = dis
    ht = dis * h0_ref[...]
    o0_ref[...] = ht[:, :_HALF]
    o1_ref[...] = ht[:, _HALF:]


_tc_prep = pl.pallas_call(
    _prep_body,
    grid=(_N // _R,),
    in_specs=[
        pl.BlockSpec((_R, _D), lambda i: (i, 0)),
        pl.BlockSpec((_R, _D), lambda i: (i, 0)),
    ],
    out_specs=[pl.BlockSpec((_R, 1), lambda i: (i, 0)), _HALF_SPEC, _HALF_SPEC],
    out_shape=[jax.ShapeDtypeStruct((_N, 1), jnp.float32),
               _HALF_SHAPE, _HALF_SHAPE],
)


def _mid_body(s_ref, t0_ref, t1_ref, h0_ref, dis_ref, w_ref, o0_ref, o1_ref,
              *, beta):
    dis = dis_ref[...]
    ht = jnp.concatenate([t0_ref[...], t1_ref[...]], axis=1)
    m = dis * (s_ref[...] + ht)
    mm = m * (1.0 - _ALPHA) + _ALPHA * h0_ref[...]
    t = beta * jnp.dot(mm, w_ref[...], preferred_element_type=jnp.float32) \
        + (1.0 - beta) * mm
    hd = jnp.maximum(t, 0.0) * dis
    o0_ref[...] = hd[:, :_HALF]
    o1_ref[...] = hd[:, _HALF:]


def _make_mid(beta):
    return pl.pallas_call(
        functools.partial(_mid_body, beta=beta),
        grid=(_N // _R,),
        in_specs=[
            pl.BlockSpec((_R, _D), lambda i: (i, 0)),
            _HALF_SPEC, _HALF_SPEC,
            pl.BlockSpec((_R, _D), lambda i: (i, 0)),
            pl.BlockSpec((_R, 1), lambda i: (i, 0)),
            pl.BlockSpec((_D, _D), lambda i: (0, 0)),
        ],
        out_specs=[_HALF_SPEC, _HALF_SPEC],
        out_shape=[_HALF_SHAPE, _HALF_SHAPE],
    )


def _fin_body(s_ref, t0_ref, t1_ref, h0_ref, dis_ref, w_ref, w1_ref, b1_ref,
              o_ref, *, beta):
    dis = dis_ref[...]
    ht = jnp.concatenate([t0_ref[...], t1_ref[...]], axis=1)
    m = dis * (s_ref[...] + ht)
    mm = m * (1.0 - _ALPHA) + _ALPHA * h0_ref[...]
    t = beta * jnp.dot(mm, w_ref[...], preferred_element_type=jnp.float32) \
        + (1.0 - beta) * mm
    h = jnp.maximum(t, 0.0)
    z = jnp.dot(h, w1_ref[...], preferred_element_type=jnp.float32) + b1_ref[...]
    colid = lax.broadcasted_iota(jnp.int32, z.shape, 1)
    zm = jnp.where(colid < 5, z, -1e30)
    mx = jnp.max(zm, axis=1, keepdims=True)
    lse = jnp.log(jnp.sum(jnp.exp(zm - mx), axis=1, keepdims=True)) + mx
    o_ref[...] = z - lse


def _make_fin(beta):
    return pl.pallas_call(
        functools.partial(_fin_body, beta=beta),
        grid=(_N // _R,),
        in_specs=[
            pl.BlockSpec((_R, _D), lambda i: (i, 0)),
            _HALF_SPEC, _HALF_SPEC,
            pl.BlockSpec((_R, _D), lambda i: (i, 0)),
            pl.BlockSpec((_R, 1), lambda i: (i, 0)),
            pl.BlockSpec((_D, _D), lambda i: (0, 0)),
            pl.BlockSpec((_D, _HALF), lambda i: (0, 0)),
            pl.BlockSpec((1, _HALF), lambda i: (0, 0)),
        ],
        out_specs=pl.BlockSpec((_R, _HALF), lambda i: (i, 0)),
        out_shape=jax.ShapeDtypeStruct((_N, _HALF), jnp.float32),
    )


_BETAS = [float(np.log(_THETA / (l + 1) + 1.0)) for l in range(_L)]
_tc_mid = [_make_mid(b) for b in _BETAS[:-1]]
_tc_fin = _make_fin(_BETAS[-1])


def kernel(x, edge_index, W0, b0, Wc, W1, b1):
    row = edge_index[0].astype(jnp.int32)
    col = edge_index[1].astype(jnp.int32)
    pad = _EPAD - _E
    rowp = jnp.concatenate([row, jnp.zeros((pad,), jnp.int32)]).reshape(_NT, _B, _EB)
    colp = jnp.concatenate([col, jnp.full((pad,), _N, jnp.int32)]).reshape(_NT, _B, _EB)
    zeros = jnp.zeros((_EB, _HALF), jnp.float32)
    ones = jnp.ones((_EB, _HALF), jnp.float32)

    cnt = _sc_count(colp, ones, zeros)
    h0 = _tc_in(x, W0, b0[None, :])
    dis, t0, t1 = _tc_prep(cnt, h0)

    for l in range(_L - 1):
        s = _sc_scatter(t0, t1, rowp, colp, zeros)
        t0, t1 = _tc_mid[l](s, t0, t1, h0, dis, Wc[l])
    s = _sc_scatter(t0, t1, rowp, colp, zeros)
    w1p = jnp.pad(W1, ((0, 0), (0, _HALF - 5)))
    b1p = jnp.pad(b1, (0, _HALF - 5))[None, :]
    out = _tc_fin(s, t0, t1, h0, dis, Wc[_L - 1], w1p, b1p)
    return out[:, :5]


# speedup vs baseline: 1.0005x; 1.0005x over previous
import functools

import jax
import jax.numpy as jnp
from jax import lax
from jax.experimental import pallas as pl
from jax.experimental.pallas import tpu as pltpu
from jax.experimental.pallas import tpu_sc as plsc
import numpy as np

_N = 10000
_D = 256
_HALF = 128
_E = 160000
_L = 8
_ALPHA = 0.1
_THETA = 0.5

_NT = 16
_EB = 128
_B = 79
_EPAD = _NT * _B * _EB

_PR = 5000
_TROWS = 5120
_ZRT = _TROWS // _NT
_DUMMY = 5056
_COPY = 312

_R = 1000

_MESH = plsc.VectorSubcoreMesh(core_axis_name="c", subcore_axis_name="s")


def _zero_table(table, zbuf, s):
    zb = s * _ZRT
    for k in range(_ZRT // _EB):
        pltpu.sync_copy(zbuf, table.at[pl.ds(zb + k * _EB, _EB)])
    zrem = _ZRT % _EB
    if zrem:
        pltpu.sync_copy(zbuf.at[pl.ds(0, zrem)],
                        table.at[pl.ds(zb + (_ZRT // _EB) * _EB, zrem)])


def _copy_out_pass(table, buf, out_hbm, c, s, p):
    def copy_out(lr0, nr):
        pltpu.sync_copy(table.at[pl.ds(lr0, nr)], buf.at[pl.ds(0, nr)])
        pltpu.sync_copy(
            buf.at[pl.ds(0, nr)],
            out_hbm.at[pl.ds(p * _PR + lr0, nr), pl.ds(c * _HALF, _HALF)])

    ob = s * _COPY
    for k in range(_COPY // _EB):
        copy_out(ob + k * _EB, _EB)
    crem = _COPY % _EB
    if crem:
        copy_out(ob + (_COPY // _EB) * _EB, crem)

    @pl.when(s == _NT - 1)
    def _():
        copy_out(_NT * _COPY, _PR - _NT * _COPY)


@functools.partial(
    pl.kernel,
    out_type=jax.ShapeDtypeStruct((_N, _D), jnp.float32),
    mesh=_MESH,
    scratch_types=[
        pltpu.VMEM((_B, _EB), jnp.int32),
        pltpu.VMEM((_B, _EB), jnp.int32),
        pltpu.VMEM((1, _EB), jnp.int32),
        pltpu.VMEM((_EB, _HALF), jnp.float32),
        pltpu.VMEM((_EB, _HALF), jnp.float32),
        pltpu.VMEM((_EB, _HALF), jnp.float32),
        pltpu.SemaphoreType.DMA,
        pltpu.SemaphoreType.DMA,
        pltpu.VMEM_SHARED((_TROWS, _HALF), jnp.float32),
    ],
)
def _sc_scatter(ht0, ht1, rowi, coli, zeros_hbm, out_hbm,
                row_v, col_v, loc_v, buf0, buf1, zbuf, g0, g1, table):
    c = lax.axis_index("c")
    s = lax.axis_index("s")
    bufs = (buf0, buf1)
    gsems = (g0, g1)

    pltpu.sync_copy(rowi.at[s], row_v)
    pltpu.sync_copy(coli.at[s], col_v)
    pltpu.sync_copy(zeros_hbm, zbuf)

    for p in range(2):
        _zero_table(table, zbuf, s)
        plsc.subcore_barrier()

        def run(h_hbm, p=p):
            def gather_start(k, t):
                pltpu.async_copy(h_hbm.at[row_v.at[t]], bufs[k], gsems[k])

            def compute_loc(t):
                for g in range(_EB // 16):
                    cv = col_v[t, pl.ds(g * 16, 16)]
                    loc = cv - (p * _PR)
                    ok = (loc >= 0) & (loc < _PR)
                    loc_v[0, pl.ds(g * 16, 16)] = jnp.where(ok, loc, _DUMMY)

            def finish(k, t):
                compute_loc(t)
                pltpu.make_async_copy(zeros_hbm, bufs[k], gsems[k]).wait()
                pltpu.sync_copy(bufs[k], table.at[loc_v.at[0]], add=True)

            gather_start(0, 0)

            def steady(o, carry):
                for b in range(2):
                    t = 2 * o + b
                    gather_start(1 - b, t + 1)
                    finish(b, t)
                return carry

            lax.fori_loop(0, (_B - 1) // 2, steady, 0)
            finish((_B - 1) % 2, _B - 1)

        @pl.when(c == 0)
        def _(run=run):
            run(ht0)

        @pl.when(c == 1)
        def _(run=run):
            run(ht1)

        plsc.subcore_barrier()
        _copy_out_pass(table, buf0, out_hbm, c, s, p)
        plsc.subcore_barrier()


@functools.partial(
    pl.kernel,
    out_type=jax.ShapeDtypeStruct((_N, _D), jnp.float32),
    mesh=_MESH,
    scratch_types=[
        pltpu.VMEM((_B, _EB), jnp.int32),
        pltpu.VMEM((1, _EB), jnp.int32),
        pltpu.VMEM((_EB, _HALF), jnp.float32),
        pltpu.VMEM((_EB, _HALF), jnp.float32),
        pltpu.VMEM_SHARED((_TROWS, _HALF), jnp.float32),
    ],
)
def _sc_count(coli, ones_hbm, zeros_hbm, out_hbm,
              col_v, loc_v, obuf, zbuf, table):
    c = lax.axis_index("c")
    s = lax.axis_index("s")

    pltpu.sync_copy(coli.at[s], col_v)
    pltpu.sync_copy(zeros_hbm, zbuf)
    pltpu.sync_copy(ones_hbm, obuf)

    for p in range(2):
        _zero_table(table, zbuf, s)
        plsc.subcore_barrier()

        def step(j, carry, p=p):
            for g in range(_EB // 16):
                cv = col_v[j, pl.ds(g * 16, 16)]
                loc = cv - (p * _PR)
                ok = (loc >= 0) & (loc < _PR)
                loc_v[0, pl.ds(g * 16, 16)] = jnp.where(ok, loc, _DUMMY)
            pltpu.sync_copy(obuf, table.at[loc_v.at[0]], add=True)
            return carry
        lax.fori_loop(0, _B, step, 0)

        plsc.subcore_barrier()
        _copy_out_pass(table, obuf, out_hbm, c, s, p)
        pltpu.sync_copy(ones_hbm, obuf)
        plsc.subcore_barrier()
        plsc.subcore_barrier()


def _in_body(x_ref, w_ref, b_ref, o_ref):
    o_ref[...] = jnp.maximum(
        jnp.dot(x_ref[...], w_ref[...], preferred_element_type=jnp.float32)
        + b_ref[...], 0.0)


_tc_in = pl.pallas_call(
    _in_body,
    grid=(_N // _R,),
    in_specs=[
        pl.BlockSpec((_R, _D), lambda i: (i, 0)),
        pl.BlockSpec((_D, _D), lambda i: (0, 0)),
        pl.BlockSpec((1, _D), lambda i: (0, 0)),
    ],
    out_specs=pl.BlockSpec((_R, _D), lambda i: (i, 0)),
    out_shape=jax.ShapeDtypeStruct((_N, _D), jnp.float32),
)

_HALF_SPEC = pl.BlockSpec((_R, _HALF), lambda i: (i, 0))
_HALF_SHAPE = jax.ShapeDtypeStruct((_N, _HALF), jnp.float32)


def _prep_body(cnt_ref, h0_ref, dis_ref, o0_ref, o1_ref):
    dis = lax.rsqrt(1.0 + cnt_ref[:, :1])
    dis_ref[...] = dis
    ht = dis * h0_ref[...]
    o0_ref[...] = ht[:, :_HALF]
    o1_ref[...] = ht[:, _HALF:]


_tc_prep = pl.pallas_call(
    _prep_body,
    grid=(_N // _R,),
    in_specs=[
        pl.BlockSpec((_R, _D), lambda i: (i, 0)),
        pl.BlockSpec((_R, _D), lambda i: (i, 0)),
    ],
    out_specs=[pl.BlockSpec((_R, 1), lambda i: (i, 0)), _HALF_SPEC, _HALF_SPEC],
    out_shape=[jax.ShapeDtypeStruct((_N, 1), jnp.float32),
               _HALF_SHAPE, _HALF_SHAPE],
)


def _mid_body(s_ref, t0_ref, t1_ref, h0_ref, dis_ref, w_ref, o0_ref, o1_ref,
              *, beta):
    dis = dis_ref[...]
    ht = jnp.concatenate([t0_ref[...], t1_ref[...]], axis=1)
    m = dis * (s_ref[...] + ht)
    mm = m * (1.0 - _ALPHA) + _ALPHA * h0_ref[...]
    t = beta * jnp.dot(mm, w_ref[...], preferred_element_type=jnp.float32) \
        + (1.0 - beta) * mm
    hd = jnp.maximum(t, 0.0) * dis
    o0_ref[...] = hd[:, :_HALF]
    o1_ref[...] = hd[:, _HALF:]


def _make_mid(beta):
    return pl.pallas_call(
        functools.partial(_mid_body, beta=beta),
        grid=(_N // _R,),
        in_specs=[
            pl.BlockSpec((_R, _D), lambda i: (i, 0)),
            _HALF_SPEC, _HALF_SPEC,
            pl.BlockSpec((_R, _D), lambda i: (i, 0)),
            pl.BlockSpec((_R, 1), lambda i: (i, 0)),
            pl.BlockSpec((_D, _D), lambda i: (0, 0)),
        ],
        out_specs=[_HALF_SPEC, _HALF_SPEC],
        out_shape=[_HALF_SHAPE, _HALF_SHAPE],
    )


def _fin_body(s_ref, t0_ref, t1_ref, h0_ref, dis_ref, w_ref, w1_ref, b1_ref,
              o_ref, *, beta):
    dis = dis_ref[...]
    ht = jnp.concatenate([t0_ref[...], t1_ref[...]], axis=1)
    m = dis * (s_ref[...] + ht)
    mm = m * (1.0 - _ALPHA) + _ALPHA * h0_ref[...]
    t = beta * jnp.dot(mm, w_ref[...], preferred_element_type=jnp.float32) \
        + (1.0 - beta) * mm
    h = jnp.maximum(t, 0.0)
    z = jnp.dot(h, w1_ref[...], preferred_element_type=jnp.float32) + b1_ref[...]
    colid = lax.broadcasted_iota(jnp.int32, z.shape, 1)
    zm = jnp.where(colid < 5, z, -1e30)
    mx = jnp.max(zm, axis=1, keepdims=True)
    lse = jnp.log(jnp.sum(jnp.exp(zm - mx), axis=1, keepdims=True)) + mx
    o_ref[...] = z - lse


def _make_fin(beta):
    return pl.pallas_call(
        functools.partial(_fin_body, beta=beta),
        grid=(_N // _R,),
        in_specs=[
            pl.BlockSpec((_R, _D), lambda i: (i, 0)),
            _HALF_SPEC, _HALF_SPEC,
            pl.BlockSpec((_R, _D), lambda i: (i, 0)),
            pl.BlockSpec((_R, 1), lambda i: (i, 0)),
            pl.BlockSpec((_D, _D), lambda i: (0, 0)),
            pl.BlockSpec((_D, _HALF), lambda i: (0, 0)),
            pl.BlockSpec((1, _HALF), lambda i: (0, 0)),
        ],
        out_specs=pl.BlockSpec((_R, _HALF), lambda i: (i, 0)),
        out_shape=jax.ShapeDtypeStruct((_N, _HALF), jnp.float32),
    )


_BETAS = [float(np.log(_THETA / (l + 1) + 1.0)) for l in range(_L)]
_tc_mid = [_make_mid(b) for b in _BETAS[:-1]]
_tc_fin = _make_fin(_BETAS[-1])


def kernel(x, edge_index, W0, b0, Wc, W1, b1):
    row = edge_index[0].astype(jnp.int32)
    col = edge_index[1].astype(jnp.int32)
    pad = _EPAD - _E
    rowp = jnp.concatenate([row, jnp.zeros((pad,), jnp.int32)]).reshape(_NT, _B, _EB)
    colp = jnp.concatenate([col, jnp.full((pad,), _N, jnp.int32)]).reshape(_NT, _B, _EB)
    zeros = jnp.zeros((_EB, _HALF), jnp.float32)
    ones = jnp.ones((_EB, _HALF), jnp.float32)

    cnt = _sc_count(colp, ones, zeros)
    h0 = _tc_in(x, W0, b0[None, :])
    dis, t0, t1 = _tc_prep(cnt, h0)

    for l in range(_L - 1):
        s = _sc_scatter(t0, t1, rowp, colp, zeros)
        t0, t1 = _tc_mid[l](s, t0, t1, h0, dis, Wc[l])
    s = _sc_scatter(t0, t1, rowp, colp, zeros)
    w1p = jnp.pad(W1, ((0, 0), (0, _HALF - 5)))
    b1p = jnp.pad(b1, (0, _HALF - 5))[None, :]
    out = _tc_fin(s, t0, t1, h0, dis, Wc[_L - 1], w1p, b1p)
    return out[:, :5]
